# initial kernel scaffold (unmeasured)
import jax
import jax.numpy as jnp
from jax import lax
from jax.experimental import pallas as pl
from jax.experimental.pallas import tpu as pltpu

M = 2048
D = 2048


def kernel(partial, resid, gamma):
    def body(partial_ref, resid_ref, gamma_ref, out_ref,
             comm_ref, send_sem, recv_sem):
        my_x = lax.axis_index("x")
        my_y = lax.axis_index("y")
        my_z = lax.axis_index("z")
        peer = (my_x, 1 - my_y, my_z)

        barrier_sem = pltpu.get_barrier_semaphore()
        pl.semaphore_signal(
            barrier_sem, inc=1,
            device_id=peer, device_id_type=pl.DeviceIdType.MESH,
        )
        pl.semaphore_wait(barrier_sem, 1)

        rdma = pltpu.make_async_remote_copy(
            src_ref=partial_ref.at[0],
            dst_ref=comm_ref,
            send_sem=send_sem,
            recv_sem=recv_sem,
            device_id=peer,
            device_id_type=pl.DeviceIdType.MESH,
        )
        rdma.start()
        rdma.wait()

        y = partial_ref[0] + comm_ref[...] + resid_ref[...]
        rms = jnp.sqrt(jnp.mean(y * y, axis=-1, keepdims=True) + 1e-6)
        out_ref[...] = y / rms * gamma_ref[...]

    return pl.pallas_call(
        body,
        out_shape=jax.ShapeDtypeStruct((M, D), jnp.float32),
        in_specs=[
            pl.BlockSpec(memory_space=pltpu.VMEM),
            pl.BlockSpec(memory_space=pltpu.VMEM),
            pl.BlockSpec(memory_space=pltpu.VMEM),
        ],
        out_specs=pl.BlockSpec(memory_space=pltpu.VMEM),
        scratch_shapes=[
            pltpu.VMEM((M, D), jnp.float32),
            pltpu.SemaphoreType.DMA,
            pltpu.SemaphoreType.DMA,
        ],
        compiler_params=pltpu.CompilerParams(collective_id=0),
    )(partial, resid, gamma.reshape(1, D))


# baseline (device time: 213790 ns/iter reference)
import jax
import jax.numpy as jnp
from jax import lax
from jax.experimental import pallas as pl
from jax.experimental.pallas import tpu as pltpu

M = 2048
D = 2048
R = 256


def _exchange(partial):

    def body(partial_ref, comm_ref, send_sem, recv_sem):
        my_x = lax.axis_index("x")
        my_y = lax.axis_index("y")
        my_z = lax.axis_index("z")
        peer = (my_x, 1 - my_y, my_z)

        barrier_sem = pltpu.get_barrier_semaphore()
        pl.semaphore_signal(
            barrier_sem, inc=1,
            device_id=peer, device_id_type=pl.DeviceIdType.MESH,
        )
        pl.semaphore_wait(barrier_sem, 1)

        rdma = pltpu.make_async_remote_copy(
            src_ref=partial_ref.at[0],
            dst_ref=comm_ref,
            send_sem=send_sem,
            recv_sem=recv_sem,
            device_id=peer,
            device_id_type=pl.DeviceIdType.MESH,
        )
        rdma.start()
        rdma.wait()

    return pl.pallas_call(
        body,
        out_shape=jax.ShapeDtypeStruct((M, D), jnp.float32),
        in_specs=[pl.BlockSpec(memory_space=pltpu.MemorySpace.HBM)],
        out_specs=pl.BlockSpec(memory_space=pltpu.MemorySpace.HBM),
        scratch_shapes=[
            pltpu.SemaphoreType.DMA,
            pltpu.SemaphoreType.DMA,
        ],
        compiler_params=pltpu.CompilerParams(collective_id=0),
    )(partial)


def _fused_ln(partial, other, resid, gamma):

    def body(partial_ref, other_ref, resid_ref, gamma_ref, out_ref):
        y = partial_ref[0] + other_ref[...] + resid_ref[...]
        rms = jnp.sqrt(jnp.mean(y * y, axis=-1, keepdims=True) + 1e-6)
        out_ref[...] = y / rms * gamma_ref[...]

    return pl.pallas_call(
        body,
        grid=(M // R,),
        out_shape=jax.ShapeDtypeStruct((M, D), jnp.float32),
        in_specs=[
            pl.BlockSpec((1, R, D), lambda i: (0, i, 0)),
            pl.BlockSpec((R, D), lambda i: (i, 0)),
            pl.BlockSpec((R, D), lambda i: (i, 0)),
            pl.BlockSpec((1, D), lambda i: (0, 0)),
        ],
        out_specs=pl.BlockSpec((R, D), lambda i: (i, 0)),
    )(partial, other, resid, gamma)


def kernel(partial, resid, gamma):
    other = _exchange(partial)
    return _fused_ln(partial, other, resid, gamma.reshape(1, D))


# device time: 132467 ns/iter; 1.6139x vs baseline; 1.6139x over previous
import jax
import jax.numpy as jnp
from jax import lax
from jax.experimental import pallas as pl
from jax.experimental.pallas import tpu as pltpu

M = 2048
D = 2048
R = 256


C = 16
CR = (M // 2) // C


def _exchange(partial):

    def body(partial_ref, other_ref, y_send, y_recv, x_send, x_recv):
        my_x = lax.axis_index("x")
        my_y = lax.axis_index("y")
        my_z = lax.axis_index("z")
        y_peer = (my_x, 1 - my_y, my_z)
        x_peer = (1 - my_x, my_y, my_z)
        half0 = my_x * (M // 2)

        barrier_sem = pltpu.get_barrier_semaphore()
        for nbr in (y_peer, x_peer):
            pl.semaphore_signal(
                barrier_sem, inc=1,
                device_id=nbr, device_id_type=pl.DeviceIdType.MESH,
            )
        pl.semaphore_wait(barrier_sem, 2)

        def chunk_rows(c):
            return pl.ds(half0 + c * CR, CR)

        y_rdma = [
            pltpu.make_async_remote_copy(
                src_ref=partial_ref.at[0, chunk_rows(c)],
                dst_ref=other_ref.at[chunk_rows(c)],
                send_sem=y_send.at[c],
                recv_sem=y_recv.at[c],
                device_id=y_peer,
                device_id_type=pl.DeviceIdType.MESH,
            )
            for c in range(C)
        ]
        x_rdma = [
            pltpu.make_async_remote_copy(
                src_ref=other_ref.at[chunk_rows(c)],
                dst_ref=other_ref.at[chunk_rows(c)],
                send_sem=x_send.at[c],
                recv_sem=x_recv.at[c],
                device_id=x_peer,
                device_id_type=pl.DeviceIdType.MESH,
            )
            for c in range(C)
        ]

        for c in range(C):
            y_rdma[c].start()
        for c in range(C):
            y_rdma[c].wait_recv()
            x_rdma[c].start()
        for c in range(C):
            x_rdma[c].wait_recv()
        for c in range(C):
            y_rdma[c].wait_send()
            x_rdma[c].wait_send()

    return pl.pallas_call(
        body,
        out_shape=jax.ShapeDtypeStruct((M, D), jnp.float32),
        in_specs=[pl.BlockSpec(memory_space=pltpu.MemorySpace.HBM)],
        out_specs=pl.BlockSpec(memory_space=pltpu.MemorySpace.HBM),
        scratch_shapes=[
            pltpu.SemaphoreType.DMA((C,)),
            pltpu.SemaphoreType.DMA((C,)),
            pltpu.SemaphoreType.DMA((C,)),
            pltpu.SemaphoreType.DMA((C,)),
        ],
        compiler_params=pltpu.CompilerParams(collective_id=0),
    )(partial)


def _fused_ln(partial, other, resid, gamma):

    def body(partial_ref, other_ref, resid_ref, gamma_ref, out_ref):
        y = partial_ref[0] + other_ref[...] + resid_ref[...]
        rms = jnp.sqrt(jnp.mean(y * y, axis=-1, keepdims=True) + 1e-6)
        out_ref[...] = y / rms * gamma_ref[...]

    return pl.pallas_call(
        body,
        grid=(M // R,),
        out_shape=jax.ShapeDtypeStruct((M, D), jnp.float32),
        in_specs=[
            pl.BlockSpec((1, R, D), lambda i: (0, i, 0)),
            pl.BlockSpec((R, D), lambda i: (i, 0)),
            pl.BlockSpec((R, D), lambda i: (i, 0)),
            pl.BlockSpec((1, D), lambda i: (0, 0)),
        ],
        out_specs=pl.BlockSpec((R, D), lambda i: (i, 0)),
    )(partial, other, resid, gamma)


def kernel(partial, resid, gamma):
    other = _exchange(partial)
    return _fused_ln(partial, other, resid, gamma.reshape(1, D))


# device time: 115682 ns/iter; 1.8481x vs baseline; 1.1451x over previous
import jax
import jax.numpy as jnp
from jax import lax
from jax.experimental import pallas as pl
from jax.experimental.pallas import tpu as pltpu

M = 2048
D = 2048
H = M // 2
C = 16
CR = H // C


def kernel(partial, resid, gamma):
    def body(partial_ref, resid_ref, gamma_ref, out_ref,
             yhalf, xhalf, pbuf, rbuf, obuf,
             y_send, y_recv, x_send, x_recv,
             pin_sem, rin_sem, out_sem):
        my_x = lax.axis_index("x")
        my_y = lax.axis_index("y")
        my_z = lax.axis_index("z")
        y_peer = (my_x, 1 - my_y, my_z)
        x_peer = (1 - my_x, my_y, my_z)
        half0 = my_x * H
        other0 = (1 - my_x) * H

        barrier_sem = pltpu.get_barrier_semaphore()
        for nbr in (y_peer, x_peer):
            pl.semaphore_signal(
                barrier_sem, inc=1,
                device_id=nbr, device_id_type=pl.DeviceIdType.MESH,
            )
        pl.semaphore_wait(barrier_sem, 2)

        y_rdma = [
            pltpu.make_async_remote_copy(
                src_ref=partial_ref.at[0, pl.ds(half0 + c * CR, CR)],
                dst_ref=yhalf.at[pl.ds(c * CR, CR)],
                send_sem=y_send.at[c],
                recv_sem=y_recv.at[c],
                device_id=y_peer,
                device_id_type=pl.DeviceIdType.MESH,
            )
            for c in range(C)
        ]
        x_rdma = [
            pltpu.make_async_remote_copy(
                src_ref=yhalf.at[pl.ds(c * CR, CR)],
                dst_ref=xhalf.at[pl.ds(c * CR, CR)],
                send_sem=x_send.at[c],
                recv_sem=x_recv.at[c],
                device_id=x_peer,
                device_id_type=pl.DeviceIdType.MESH,
            )
            for c in range(C)
        ]

        for c in range(C):
            y_rdma[c].start()

        def rows0(k):
            return half0 + k * CR if k < C else other0 + (k - C) * CR

        def stage_in(k):
            s = k % 2
            pin = pltpu.make_async_copy(
                partial_ref.at[0, pl.ds(rows0(k), CR)], pbuf.at[s],
                pin_sem.at[s],
            )
            rin = pltpu.make_async_copy(
                resid_ref.at[pl.ds(rows0(k), CR)], rbuf.at[s],
                rin_sem.at[s],
            )
            pin.start()
            rin.start()
            return pin, rin

        out_copies = [None] * (2 * C)
        stage = stage_in(0)
        for k in range(2 * C):
            if k < C:
                y_rdma[k].wait_recv()
                x_rdma[k].start()
                comm = yhalf.at[pl.ds(k * CR, CR)]
            else:
                x_rdma[k - C].wait_recv()
                comm = xhalf.at[pl.ds((k - C) * CR, CR)]

            pin, rin = stage
            if k + 1 < 2 * C:
                stage = stage_in(k + 1)
            pin.wait()
            rin.wait()
            s = k % 2
            if k >= 2:
                out_copies[k - 2].wait()

            y = pbuf[s] + comm[...] + rbuf[s]
            rms = jnp.sqrt(jnp.mean(y * y, axis=-1, keepdims=True) + 1e-6)
            obuf[s] = y / rms * gamma_ref[...]

            oc = pltpu.make_async_copy(
                obuf.at[s], out_ref.at[pl.ds(rows0(k), CR)], out_sem.at[s],
            )
            oc.start()
            out_copies[k] = oc

        out_copies[2 * C - 2].wait()
        out_copies[2 * C - 1].wait()
        for c in range(C):
            y_rdma[c].wait_send()
            x_rdma[c].wait_send()

    return pl.pallas_call(
        body,
        out_shape=jax.ShapeDtypeStruct((M, D), jnp.float32),
        in_specs=[
            pl.BlockSpec(memory_space=pltpu.MemorySpace.HBM),
            pl.BlockSpec(memory_space=pltpu.MemorySpace.HBM),
            pl.BlockSpec(memory_space=pltpu.VMEM),
        ],
        out_specs=pl.BlockSpec(memory_space=pltpu.MemorySpace.HBM),
        scratch_shapes=[
            pltpu.VMEM((H, D), jnp.float32),
            pltpu.VMEM((H, D), jnp.float32),
            pltpu.VMEM((2, CR, D), jnp.float32),
            pltpu.VMEM((2, CR, D), jnp.float32),
            pltpu.VMEM((2, CR, D), jnp.float32),
            pltpu.SemaphoreType.DMA((C,)),
            pltpu.SemaphoreType.DMA((C,)),
            pltpu.SemaphoreType.DMA((C,)),
            pltpu.SemaphoreType.DMA((C,)),
            pltpu.SemaphoreType.DMA((2,)),
            pltpu.SemaphoreType.DMA((2,)),
            pltpu.SemaphoreType.DMA((2,)),
        ],
        compiler_params=pltpu.CompilerParams(collective_id=0),
    )(partial, resid, gamma.reshape(1, D))


# device time: 109576 ns/iter; 1.9511x vs baseline; 1.0557x over previous
import jax
import jax.numpy as jnp
from jax import lax
from jax.experimental import pallas as pl
from jax.experimental.pallas import tpu as pltpu

M = 2048
D = 2048
H = M // 2
C = 16
CR = H // C


def kernel(partial, resid, gamma):
    def body(partial_ref, resid_ref, gamma_ref, out_ref,
             yhalf, xhalf, pbuf, rbuf, obuf,
             y_send, y_recv, x_send, x_recv,
             pin_sem, rin_sem, out_sem):
        my_x = lax.axis_index("x")
        my_y = lax.axis_index("y")
        my_z = lax.axis_index("z")
        y_peer = (my_x, 1 - my_y, my_z)
        x_peer = (1 - my_x, my_y, my_z)
        half0 = my_x * H
        other0 = (1 - my_x) * H

        barrier_sem = pltpu.get_barrier_semaphore()
        for nbr in (y_peer, x_peer):
            pl.semaphore_signal(
                barrier_sem, inc=1,
                device_id=nbr, device_id_type=pl.DeviceIdType.MESH,
            )
        pl.semaphore_wait(barrier_sem, 2)

        y_rdma = [
            pltpu.make_async_remote_copy(
                src_ref=partial_ref.at[0, pl.ds(half0 + c * CR, CR)],
                dst_ref=yhalf.at[pl.ds(c * CR, CR)],
                send_sem=y_send.at[c],
                recv_sem=y_recv.at[c],
                device_id=y_peer,
                device_id_type=pl.DeviceIdType.MESH,
            )
            for c in range(C)
        ]
        x_rdma = [
            pltpu.make_async_remote_copy(
                src_ref=yhalf.at[pl.ds(c * CR, CR)],
                dst_ref=xhalf.at[pl.ds(c * CR, CR)],
                send_sem=x_send.at[c],
                recv_sem=x_recv.at[c],
                device_id=x_peer,
                device_id_type=pl.DeviceIdType.MESH,
            )
            for c in range(C)
        ]

        for c in range(C):
            y_rdma[c].start()

        order = [("y", 0)]
        for c in range(1, C):
            order.append(("y", c))
            order.append(("x", c - 1))
        order.append(("x", C - 1))

        def rows0(kind, c):
            return (half0 if kind == "y" else other0) + c * CR

        def stage_in(j):
            kind, c = order[j]
            s = j % 2
            pin = pltpu.make_async_copy(
                partial_ref.at[0, pl.ds(rows0(kind, c), CR)], pbuf.at[s],
                pin_sem.at[s],
            )
            rin = pltpu.make_async_copy(
                resid_ref.at[pl.ds(rows0(kind, c), CR)], rbuf.at[s],
                rin_sem.at[s],
            )
            pin.start()
            rin.start()
            return pin, rin

        out_copies = [None] * (2 * C)
        stage = stage_in(0)
        for j in range(2 * C):
            kind, c = order[j]
            if kind == "y":
                y_rdma[c].wait_recv()
                x_rdma[c].start()
                comm = yhalf.at[pl.ds(c * CR, CR)]
            else:
                x_rdma[c].wait_recv()
                comm = xhalf.at[pl.ds(c * CR, CR)]

            pin, rin = stage
            if j + 1 < 2 * C:
                stage = stage_in(j + 1)
            pin.wait()
            rin.wait()
            s = j % 2
            if j >= 2:
                out_copies[j - 2].wait()

            y = pbuf[s] + comm[...] + rbuf[s]
            inv = lax.rsqrt(jnp.mean(y * y, axis=-1, keepdims=True) + 1e-6)
            obuf[s] = y * inv * gamma_ref[...]

            oc = pltpu.make_async_copy(
                obuf.at[s], out_ref.at[pl.ds(rows0(kind, c), CR)],
                out_sem.at[s],
            )
            oc.start()
            out_copies[j] = oc

        out_copies[2 * C - 2].wait()
        out_copies[2 * C - 1].wait()
        for c in range(C):
            y_rdma[c].wait_send()
            x_rdma[c].wait_send()

    return pl.pallas_call(
        body,
        out_shape=jax.ShapeDtypeStruct((M, D), jnp.float32),
        in_specs=[
            pl.BlockSpec(memory_space=pltpu.MemorySpace.HBM),
            pl.BlockSpec(memory_space=pltpu.MemorySpace.HBM),
            pl.BlockSpec(memory_space=pltpu.VMEM),
        ],
        out_specs=pl.BlockSpec(memory_space=pltpu.MemorySpace.HBM),
        scratch_shapes=[
            pltpu.VMEM((H, D), jnp.float32),
            pltpu.VMEM((H, D), jnp.float32),
            pltpu.VMEM((2, CR, D), jnp.float32),
            pltpu.VMEM((2, CR, D), jnp.float32),
            pltpu.VMEM((2, CR, D), jnp.float32),
            pltpu.SemaphoreType.DMA((C,)),
            pltpu.SemaphoreType.DMA((C,)),
            pltpu.SemaphoreType.DMA((C,)),
            pltpu.SemaphoreType.DMA((C,)),
            pltpu.SemaphoreType.DMA((2,)),
            pltpu.SemaphoreType.DMA((2,)),
            pltpu.SemaphoreType.DMA((2,)),
        ],
        compiler_params=pltpu.CompilerParams(collective_id=0),
    )(partial, resid, gamma.reshape(1, D))
